# baseline (speedup 1.0000x reference)
import functools

import jax
import jax.numpy as jnp
from jax import lax
from jax.experimental import pallas as pl
from jax.experimental.pallas import tpu as pltpu
from jax.experimental.pallas import tpu_sc as plsc

N_NODES = 10000
N_GRAPHS = 64
D = 128
E = 320000
BN_EPS = 1e-5

NC = 2
NS = 16
NW = NC * NS
EPW = E // NW
CH = 96
NCHUNK = EPW // CH
TAIL = EPW - NCHUNK * CH
ROWS_PER_TILE = 632
NPAD = ROWS_PER_TILE * NS

_mesh = plsc.VectorSubcoreMesh(
    core_axis_name="c", subcore_axis_name="s", num_cores=NC, num_subcores=NS
)


NBR = 3
NBI = 6
GD = 2
ID = 5
SW = NBR - GD
NSTEADY = (NCHUNK - 1 - ID - 2 + 1) // NBI
STEADY_END = 2 + NBI * NSTEADY


@functools.partial(
    pl.kernel,
    mesh=_mesh,
    out_type=jax.ShapeDtypeStruct((NC, NPAD, D), jnp.float32),
    scratch_types=(
        [pltpu.VMEM((CH,), jnp.int32) for _ in range(2 * NBI)]
        + [pltpu.VMEM((CH, D), jnp.float32) for _ in range(NBR)]
        + [pltpu.VMEM((TAIL,), jnp.int32) for _ in range(2)]
        + [pltpu.VMEM((TAIL, D), jnp.float32)]
        + [pltpu.VMEM_SHARED((NPAD, D), jnp.float32)]
        + [pltpu.SemaphoreType.DMA for _ in range(NBI + 2 * NBR + 1)]
    ),
)
def _sc_segsum(h_hbm, src_hbm, dst_hbm, zeros_hbm, out_hbm, *refs):
    srcs = refs[0:NBI]
    dsts = refs[NBI:2 * NBI]
    rows = refs[2 * NBI:2 * NBI + NBR]
    src_t, dst_t, rows_t = refs[2 * NBI + NBR:2 * NBI + NBR + 3]
    agg_sh = refs[2 * NBI + NBR + 3]
    sems = refs[2 * NBI + NBR + 4:]
    isems = sems[0:NBI]
    gsems = sems[NBI:NBI + NBR]
    ssems = sems[NBI + NBR:NBI + 2 * NBR]
    tsem = sems[NBI + 2 * NBR]

    c = lax.axis_index("c")
    s = lax.axis_index("s")
    wid = s * NC + c

    row0 = pl.multiple_of(s * ROWS_PER_TILE, 8)
    pltpu.sync_copy(zeros_hbm.at[pl.ds(row0, ROWS_PER_TILE)],
                    agg_sh.at[pl.ds(row0, ROWS_PER_TILE)])
    plsc.subcore_barrier()

    base = wid * EPW

    def idx_start(i, b):
        off = pl.multiple_of(base + i * CH, 8)
        pltpu.async_copy(src_hbm.at[pl.ds(off, CH)], srcs[b], isems[b])
        pltpu.async_copy(dst_hbm.at[pl.ds(off, CH)], dsts[b], isems[b])

    def idx_wait(b):
        pltpu.make_async_copy(src_hbm.at[pl.ds(0, CH)], srcs[b], isems[b]).wait()
        pltpu.make_async_copy(dst_hbm.at[pl.ds(0, CH)], dsts[b], isems[b]).wait()

    CH2 = CH // 2

    def gather_start(bi, br):
        pltpu.async_copy(h_hbm.at[srcs[bi].at[pl.ds(0, CH2)]],
                         rows[br].at[pl.ds(0, CH2)], gsems[br])
        pltpu.async_copy(h_hbm.at[srcs[bi].at[pl.ds(CH2, CH2)]],
                         rows[br].at[pl.ds(CH2, CH2)], gsems[br])

    def gather_wait(bi, br):
        pltpu.make_async_copy(h_hbm.at[srcs[bi].at[pl.ds(0, CH2)]],
                              rows[br].at[pl.ds(0, CH2)], gsems[br]).wait()
        pltpu.make_async_copy(h_hbm.at[srcs[bi].at[pl.ds(CH2, CH2)]],
                              rows[br].at[pl.ds(CH2, CH2)], gsems[br]).wait()

    def scatter_start(bi, br):
        pltpu.async_copy(rows[br], agg_sh.at[dsts[bi]], ssems[br], add=True)

    def scatter_wait(bi, br):
        pltpu.make_async_copy(rows[br], agg_sh.at[dsts[bi]], ssems[br]).wait()

    def emit_body(i, r, first=False, last_g=True, last_i=True):
        if last_g:
            idx_wait((r + GD) % NBI)
        if not first:
            scatter_wait((r + GD) % NBI, (r + GD) % NBR)
        if last_g:
            gather_start((r + GD) % NBI, (r + GD) % NBR)
        gather_wait(r, r % NBR)
        scatter_start(r, r % NBR)
        if last_i:
            idx_start(i + ID, (r + ID) % NBI)

    for j in range(ID):
        idx_start(j, j)
    for j in range(GD):
        idx_wait(j)
        gather_start(j, j)
    for j in range(2):
        emit_body(j, j, first=(j < SW))

    def outer(g, carry):
        i0 = 2 + NBI * g
        for k in range(NBI):
            emit_body(i0 + k, (2 + k) % NBI)
        return carry

    lax.fori_loop(0, NSTEADY, outer, 0)

    for i in range(STEADY_END, NCHUNK):
        emit_body(i, i % NBI, last_g=(i + GD < NCHUNK), last_i=(i + ID < NCHUNK))
    for i in range(NCHUNK - SW, NCHUNK):
        scatter_wait(i % NBI, i % NBR)

    toff = pl.multiple_of(base + NCHUNK * CH, 8)
    pltpu.sync_copy(src_hbm.at[pl.ds(toff, TAIL)], src_t)
    pltpu.sync_copy(dst_hbm.at[pl.ds(toff, TAIL)], dst_t)
    pltpu.async_copy(h_hbm.at[src_t], rows_t, tsem).wait()
    pltpu.sync_copy(rows_t, agg_sh.at[dst_t], add=True)

    plsc.subcore_barrier()

    pltpu.sync_copy(agg_sh.at[pl.ds(row0, ROWS_PER_TILE)],
                    out_hbm.at[c, pl.ds(row0, ROWS_PER_TILE)])


R = 1000
GRID = N_NODES // R


def _mlp_stats_body(h_ref, agg_ref, w1_ref, w2_ref, p_ref, stats_ref):
    z = h_ref[...] + agg_ref[0] + agg_ref[1]
    y = jnp.maximum(jnp.dot(z, w1_ref[...], preferred_element_type=jnp.float32), 0.0)
    y = jnp.dot(y, w2_ref[...], preferred_element_type=jnp.float32)
    p = jnp.maximum(y, 0.0)
    p_ref[...] = p
    st = jnp.concatenate(
        [jnp.sum(p, axis=0, keepdims=True),
         jnp.sum(p * p, axis=0, keepdims=True)], axis=0)

    @pl.when(pl.program_id(0) == 0)
    def _():
        stats_ref[...] = jnp.zeros_like(stats_ref)

    stats_ref[...] += st


_mlp_stats = pl.pallas_call(
    _mlp_stats_body,
    grid=(GRID,),
    in_specs=[
        pl.BlockSpec((R, D), lambda i: (i, 0)),
        pl.BlockSpec((NC, R, D), lambda i: (0, i, 0)),
        pl.BlockSpec((D, D), lambda i: (0, 0)),
        pl.BlockSpec((D, D), lambda i: (0, 0)),
    ],
    out_specs=[
        pl.BlockSpec((R, D), lambda i: (i, 0)),
        pl.BlockSpec((2, D), lambda i: (0, 0)),
    ],
    out_shape=[
        jax.ShapeDtypeStruct((N_NODES, D), jnp.float32),
        jax.ShapeDtypeStruct((2, D), jnp.float32),
    ],
)


def _bn_body(p_ref, stats_ref, g_ref, b_ref, o_ref):
    inv_n = 1.0 / N_NODES
    mean = stats_ref[0] * inv_n
    var = stats_ref[1] * inv_n - mean * mean
    scale = g_ref[0] * lax.rsqrt(var + BN_EPS)
    shift = b_ref[0] - mean * scale
    o_ref[...] = p_ref[...] * scale + shift


_bn_apply = pl.pallas_call(
    _bn_body,
    grid=(GRID,),
    in_specs=[
        pl.BlockSpec((R, D), lambda i: (i, 0)),
        pl.BlockSpec((2, D), lambda i: (0, 0)),
        pl.BlockSpec((1, D), lambda i: (0, 0)),
        pl.BlockSpec((1, D), lambda i: (0, 0)),
    ],
    out_specs=pl.BlockSpec((R, D), lambda i: (i, 0)),
    out_shape=jax.ShapeDtypeStruct((N_NODES, D), jnp.float32),
)


def _final_body(h_ref, agg_ref, w1_ref, w2_ref, batch_ref, z_ref, pool_ref):
    z = h_ref[...] + agg_ref[0] + agg_ref[1]
    y = jnp.maximum(jnp.dot(z, w1_ref[...], preferred_element_type=jnp.float32), 0.0)
    y = jnp.dot(y, w2_ref[...], preferred_element_type=jnp.float32)
    z_ref[...] = y
    seg = lax.broadcasted_iota(jnp.int32, (N_GRAPHS, 1), 0)
    mask = (batch_ref[0] == seg).astype(jnp.float32)
    part = jnp.dot(mask, y, preferred_element_type=jnp.float32)

    @pl.when(pl.program_id(0) == 0)
    def _():
        pool_ref[...] = jnp.zeros_like(pool_ref)

    pool_ref[...] += part


_final_mlp_pool = pl.pallas_call(
    _final_body,
    grid=(GRID,),
    in_specs=[
        pl.BlockSpec((R, D), lambda i: (i, 0)),
        pl.BlockSpec((NC, R, D), lambda i: (0, i, 0)),
        pl.BlockSpec((D, D), lambda i: (0, 0)),
        pl.BlockSpec((D, D), lambda i: (0, 0)),
        pl.BlockSpec((1, 1, R), lambda i: (i, 0, 0)),
    ],
    out_specs=[
        pl.BlockSpec((R, D), lambda i: (i, 0)),
        pl.BlockSpec((N_GRAPHS, D), lambda i: (0, 0)),
    ],
    out_shape=[
        jax.ShapeDtypeStruct((N_NODES, D), jnp.float32),
        jax.ShapeDtypeStruct((N_GRAPHS, D), jnp.float32),
    ],
)


def kernel(x, edge_index, batch, W1_0, W2_0, g0, b0, W1_1, W2_1, g1, b1,
           W1_2, W2_2):
    src = edge_index[0].astype(jnp.int32)
    dst = edge_index[1].astype(jnp.int32)
    zeros = jnp.zeros((NPAD, D), jnp.float32)
    batch3d = batch.astype(jnp.int32).reshape(GRID, 1, R)
    g0r, b0r = g0.reshape(1, D), b0.reshape(1, D)
    g1r, b1r = g1.reshape(1, D), b1.reshape(1, D)

    h = x
    agg = _sc_segsum(h, src, dst, zeros)
    p, stats = _mlp_stats(h, agg, W1_0, W2_0)
    h1 = _bn_apply(p, stats, g0r, b0r)

    agg = _sc_segsum(h1, src, dst, zeros)
    p, stats = _mlp_stats(h1, agg, W1_1, W2_1)
    h2 = _bn_apply(p, stats, g1r, b1r)

    agg = _sc_segsum(h2, src, dst, zeros)
    z3, xpool = _final_mlp_pool(h2, agg, W1_2, W2_2, batch3d)

    return (xpool, jnp.concatenate([h1, h2, z3], axis=1))


# speedup vs baseline: 1.0207x; 1.0207x over previous
import functools

import jax
import jax.numpy as jnp
from jax import lax
from jax.experimental import pallas as pl
from jax.experimental.pallas import tpu as pltpu
from jax.experimental.pallas import tpu_sc as plsc

N_NODES = 10000
N_GRAPHS = 64
D = 128
E = 320000
BN_EPS = 1e-5

NC = 2
NS = 16
NW = NC * NS
EPW = E // NW
CH = 96
NCHUNK = EPW // CH
TAIL = EPW - NCHUNK * CH
ROWS_PER_TILE = 632
NPAD = ROWS_PER_TILE * NS

_mesh = plsc.VectorSubcoreMesh(
    core_axis_name="c", subcore_axis_name="s", num_cores=NC, num_subcores=NS
)


NBR = 3
NBI = 6
GD = 2
ID = 5
SW = NBR - GD
NSTEADY = (NCHUNK - 1 - ID - 2 + 1) // NBI
STEADY_END = 2 + NBI * NSTEADY


def _make_sc_segsum(with_deg):
  out_type = [jax.ShapeDtypeStruct((NC, NPAD, D), jnp.float32)]
  scratch = (
      [pltpu.VMEM((CH,), jnp.int32) for _ in range(2 * NBI)]
      + [pltpu.VMEM((CH, D), jnp.float32) for _ in range(NBR)]
      + [pltpu.VMEM((TAIL,), jnp.int32) for _ in range(2)]
      + [pltpu.VMEM((TAIL, D), jnp.float32)]
      + [pltpu.VMEM_SHARED((NPAD, D), jnp.float32)]
      + [pltpu.SemaphoreType.DMA for _ in range(NBI + 2 * NBR + 1)]
  )
  if with_deg:
    out_type.append(jax.ShapeDtypeStruct((NC * NPAD,), jnp.float32))
    scratch += [pltpu.VMEM((CH,), jnp.float32),
                pltpu.VMEM((ROWS_PER_TILE,), jnp.float32),
                pltpu.VMEM_SHARED((NPAD,), jnp.float32)]

  @functools.partial(pl.kernel, mesh=_mesh, out_type=out_type,
                     scratch_types=scratch)
  def sc_segsum(h_hbm, src_hbm, dst_hbm, zeros_hbm, *outs_refs):
    if with_deg:
        out_hbm, deg_hbm = outs_refs[:2]
        refs = outs_refs[2:]
        ones_v, dbuf, deg_sh = refs[-3:]
    else:
        (out_hbm,), refs = outs_refs[:1], outs_refs[1:]
    srcs = refs[0:NBI]
    dsts = refs[NBI:2 * NBI]
    rows = refs[2 * NBI:2 * NBI + NBR]
    src_t, dst_t, rows_t = refs[2 * NBI + NBR:2 * NBI + NBR + 3]
    agg_sh = refs[2 * NBI + NBR + 3]
    sems = refs[2 * NBI + NBR + 4:2 * NBI + NBR + 4 + NBI + 2 * NBR + 1]
    isems = sems[0:NBI]
    gsems = sems[NBI:NBI + NBR]
    ssems = sems[NBI + NBR:NBI + 2 * NBR]
    tsem = sems[NBI + 2 * NBR]

    c = lax.axis_index("c")
    s = lax.axis_index("s")
    wid = s * NC + c

    row0 = pl.multiple_of(s * ROWS_PER_TILE, 8)
    pltpu.sync_copy(zeros_hbm.at[pl.ds(row0, ROWS_PER_TILE)],
                    agg_sh.at[pl.ds(row0, ROWS_PER_TILE)])
    if with_deg:
        for k in range(CH // 16):
            ones_v[pl.ds(16 * k, 16)] = jnp.full((16,), 1.0, jnp.float32)
        for k in range(ROWS_PER_TILE // 16):
            dbuf[pl.ds(16 * k, 16)] = jnp.zeros((16,), jnp.float32)
        pltpu.sync_copy(dbuf, deg_sh.at[pl.ds(row0, ROWS_PER_TILE)])
    plsc.subcore_barrier()

    base = wid * EPW

    def idx_start(i, b):
        off = pl.multiple_of(base + i * CH, 8)
        pltpu.async_copy(src_hbm.at[pl.ds(off, CH)], srcs[b], isems[b])
        pltpu.async_copy(dst_hbm.at[pl.ds(off, CH)], dsts[b], isems[b])

    def idx_wait(b):
        pltpu.make_async_copy(src_hbm.at[pl.ds(0, CH)], srcs[b], isems[b]).wait()
        pltpu.make_async_copy(dst_hbm.at[pl.ds(0, CH)], dsts[b], isems[b]).wait()

    CH2 = CH // 2

    def gather_start(bi, br):
        pltpu.async_copy(h_hbm.at[srcs[bi].at[pl.ds(0, CH2)]],
                         rows[br].at[pl.ds(0, CH2)], gsems[br])
        pltpu.async_copy(h_hbm.at[srcs[bi].at[pl.ds(CH2, CH2)]],
                         rows[br].at[pl.ds(CH2, CH2)], gsems[br])

    def gather_wait(bi, br):
        pltpu.make_async_copy(h_hbm.at[srcs[bi].at[pl.ds(0, CH2)]],
                              rows[br].at[pl.ds(0, CH2)], gsems[br]).wait()
        pltpu.make_async_copy(h_hbm.at[srcs[bi].at[pl.ds(CH2, CH2)]],
                              rows[br].at[pl.ds(CH2, CH2)], gsems[br]).wait()

    def scatter_start(bi, br):
        pltpu.async_copy(rows[br], agg_sh.at[dsts[bi]], ssems[br], add=True)
        if with_deg:
            pltpu.async_copy(ones_v, deg_sh.at[dsts[bi]], ssems[br], add=True)

    def scatter_wait(bi, br):
        pltpu.make_async_copy(rows[br], agg_sh.at[dsts[bi]], ssems[br]).wait()
        if with_deg:
            pltpu.make_async_copy(ones_v, deg_sh.at[dsts[bi]],
                                  ssems[br]).wait()

    def emit_body(i, r, first=False, last_g=True, last_i=True):
        if last_g:
            idx_wait((r + GD) % NBI)
        if not first:
            scatter_wait((r + GD) % NBI, (r + GD) % NBR)
        if last_g:
            gather_start((r + GD) % NBI, (r + GD) % NBR)
        gather_wait(r, r % NBR)
        scatter_start(r, r % NBR)
        if last_i:
            idx_start(i + ID, (r + ID) % NBI)

    for j in range(ID):
        idx_start(j, j)
    for j in range(GD):
        idx_wait(j)
        gather_start(j, j)
    for j in range(2):
        emit_body(j, j, first=(j < SW))

    def outer(g, carry):
        i0 = 2 + NBI * g
        for k in range(NBI):
            emit_body(i0 + k, (2 + k) % NBI)
        return carry

    lax.fori_loop(0, NSTEADY, outer, 0)

    for i in range(STEADY_END, NCHUNK):
        emit_body(i, i % NBI, last_g=(i + GD < NCHUNK), last_i=(i + ID < NCHUNK))
    for i in range(NCHUNK - SW, NCHUNK):
        scatter_wait(i % NBI, i % NBR)

    toff = pl.multiple_of(base + NCHUNK * CH, 8)
    pltpu.sync_copy(src_hbm.at[pl.ds(toff, TAIL)], src_t)
    pltpu.sync_copy(dst_hbm.at[pl.ds(toff, TAIL)], dst_t)
    pltpu.async_copy(h_hbm.at[src_t], rows_t, tsem).wait()
    pltpu.sync_copy(rows_t, agg_sh.at[dst_t], add=True)
    if with_deg:
        pltpu.sync_copy(ones_v.at[pl.ds(0, TAIL)], deg_sh.at[dst_t], add=True)

    plsc.subcore_barrier()

    pltpu.sync_copy(agg_sh.at[pl.ds(row0, ROWS_PER_TILE)],
                    out_hbm.at[c, pl.ds(row0, ROWS_PER_TILE)])
    if with_deg:
        doff = pl.multiple_of(c * NPAD + row0, 8)
        pltpu.sync_copy(deg_sh.at[pl.ds(row0, ROWS_PER_TILE)], dbuf)
        pltpu.sync_copy(dbuf, deg_hbm.at[pl.ds(doff, ROWS_PER_TILE)])

  return sc_segsum


_sc_segsum = _make_sc_segsum(False)
_sc_segsum_deg = _make_sc_segsum(True)


R = 1000
GRID = N_NODES // R


def _mlp_stats_body(h_ref, agg_ref, w1_ref, w2_ref, p_ref, stats_ref):
    z = h_ref[...] + agg_ref[0] + agg_ref[1]
    y = jnp.maximum(jnp.dot(z, w1_ref[...], preferred_element_type=jnp.float32), 0.0)
    y = jnp.dot(y, w2_ref[...], preferred_element_type=jnp.float32)
    p = jnp.maximum(y, 0.0)
    p_ref[...] = p
    st = jnp.concatenate(
        [jnp.sum(p, axis=0, keepdims=True),
         jnp.sum(p * p, axis=0, keepdims=True)], axis=0)

    @pl.when(pl.program_id(0) == 0)
    def _():
        stats_ref[...] = jnp.zeros_like(stats_ref)

    stats_ref[...] += st


_mlp_stats = pl.pallas_call(
    _mlp_stats_body,
    grid=(GRID,),
    in_specs=[
        pl.BlockSpec((R, D), lambda i: (i, 0)),
        pl.BlockSpec((NC, R, D), lambda i: (0, i, 0)),
        pl.BlockSpec((D, D), lambda i: (0, 0)),
        pl.BlockSpec((D, D), lambda i: (0, 0)),
    ],
    out_specs=[
        pl.BlockSpec((R, D), lambda i: (i, 0)),
        pl.BlockSpec((2, D), lambda i: (0, 0)),
    ],
    out_shape=[
        jax.ShapeDtypeStruct((N_NODES, D), jnp.float32),
        jax.ShapeDtypeStruct((2, D), jnp.float32),
    ],
)


def _bn_coefs(stats_ref, g_ref, b_ref):
    inv_n = 1.0 / N_NODES
    mean = stats_ref[0] * inv_n
    var = stats_ref[1] * inv_n - mean * mean
    a = g_ref[0] * lax.rsqrt(var + BN_EPS)
    cshift = b_ref[0] - mean * a
    return a, cshift


def _mid_body(p_ref, agg_ref, stats_ref, g_ref, b_ref, d0_ref, d1_ref,
              w1_ref, w2_ref, pout_ref, stats_out_ref, h_ref):
    a, cshift = _bn_coefs(stats_ref, g_ref, b_ref)
    p = p_ref[...]
    h_ref[...] = p * a + cshift
    deg = d0_ref[...] + d1_ref[...]
    z = (p + agg_ref[0] + agg_ref[1]) * a + (1.0 + deg) * cshift
    y = jnp.maximum(jnp.dot(z, w1_ref[...], preferred_element_type=jnp.float32), 0.0)
    y = jnp.dot(y, w2_ref[...], preferred_element_type=jnp.float32)
    pn = jnp.maximum(y, 0.0)
    pout_ref[...] = pn
    st = jnp.concatenate(
        [jnp.sum(pn, axis=0, keepdims=True),
         jnp.sum(pn * pn, axis=0, keepdims=True)], axis=0)

    @pl.when(pl.program_id(0) == 0)
    def _():
        stats_out_ref[...] = jnp.zeros_like(stats_out_ref)

    stats_out_ref[...] += st


_MID_SPECS = [
    pl.BlockSpec((R, D), lambda i: (i, 0)),
    pl.BlockSpec((NC, R, D), lambda i: (0, i, 0)),
    pl.BlockSpec((2, D), lambda i: (0, 0)),
    pl.BlockSpec((1, D), lambda i: (0, 0)),
    pl.BlockSpec((1, D), lambda i: (0, 0)),
    pl.BlockSpec((R, 1), lambda i: (i, 0)),
    pl.BlockSpec((R, 1), lambda i: (i, 0)),
    pl.BlockSpec((D, D), lambda i: (0, 0)),
    pl.BlockSpec((D, D), lambda i: (0, 0)),
]

_mlp_mid = pl.pallas_call(
    _mid_body,
    grid=(GRID,),
    in_specs=_MID_SPECS,
    out_specs=[
        pl.BlockSpec((R, D), lambda i: (i, 0)),
        pl.BlockSpec((2, D), lambda i: (0, 0)),
        pl.BlockSpec((R, D), lambda i: (i, 0)),
    ],
    out_shape=[
        jax.ShapeDtypeStruct((N_NODES, D), jnp.float32),
        jax.ShapeDtypeStruct((2, D), jnp.float32),
        jax.ShapeDtypeStruct((N_NODES, D), jnp.float32),
    ],
)


def _final_body(p_ref, agg_ref, stats_ref, g_ref, b_ref, d0_ref, d1_ref,
                w1_ref, w2_ref, batch_ref, z_ref, pool_ref, h_ref):
    a, cshift = _bn_coefs(stats_ref, g_ref, b_ref)
    p = p_ref[...]
    h_ref[...] = p * a + cshift
    deg = d0_ref[...] + d1_ref[...]
    z = (p + agg_ref[0] + agg_ref[1]) * a + (1.0 + deg) * cshift
    y = jnp.maximum(jnp.dot(z, w1_ref[...], preferred_element_type=jnp.float32), 0.0)
    y = jnp.dot(y, w2_ref[...], preferred_element_type=jnp.float32)
    z_ref[...] = y
    seg = lax.broadcasted_iota(jnp.int32, (N_GRAPHS, 1), 0)
    mask = (batch_ref[0] == seg).astype(jnp.float32)
    part = jnp.dot(mask, y, preferred_element_type=jnp.float32)

    @pl.when(pl.program_id(0) == 0)
    def _():
        pool_ref[...] = jnp.zeros_like(pool_ref)

    pool_ref[...] += part


_final_mlp_pool = pl.pallas_call(
    _final_body,
    grid=(GRID,),
    in_specs=_MID_SPECS + [pl.BlockSpec((1, 1, R), lambda i: (i, 0, 0))],
    out_specs=[
        pl.BlockSpec((R, D), lambda i: (i, 0)),
        pl.BlockSpec((N_GRAPHS, D), lambda i: (0, 0)),
        pl.BlockSpec((R, D), lambda i: (i, 0)),
    ],
    out_shape=[
        jax.ShapeDtypeStruct((N_NODES, D), jnp.float32),
        jax.ShapeDtypeStruct((N_GRAPHS, D), jnp.float32),
        jax.ShapeDtypeStruct((N_NODES, D), jnp.float32),
    ],
)


def kernel(x, edge_index, batch, W1_0, W2_0, g0, b0, W1_1, W2_1, g1, b1,
           W1_2, W2_2):
    src = edge_index[0].astype(jnp.int32)
    dst = edge_index[1].astype(jnp.int32)
    zeros = jnp.zeros((NPAD, D), jnp.float32)
    batch3d = batch.astype(jnp.int32).reshape(GRID, 1, R)
    g0r, b0r = g0.reshape(1, D), b0.reshape(1, D)
    g1r, b1r = g1.reshape(1, D), b1.reshape(1, D)

    agg, deg = _sc_segsum_deg(x, src, dst, zeros)
    d0 = deg[:NPAD].reshape(NPAD, 1)
    d1 = deg[NPAD:].reshape(NPAD, 1)
    p0, stats0 = _mlp_stats(x, agg, W1_0, W2_0)

    agg, = _sc_segsum(p0, src, dst, zeros)
    p1, stats1, h1 = _mlp_mid(p0, agg, stats0, g0r, b0r, d0, d1, W1_1, W2_1)

    agg, = _sc_segsum(p1, src, dst, zeros)
    z3, xpool, h2 = _final_mlp_pool(p1, agg, stats1, g1r, b1r, d0, d1,
                                    W1_2, W2_2, batch3d)

    return (xpool, jnp.concatenate([h1, h2, z3], axis=1))


# speedup vs baseline: 1.0397x; 1.0186x over previous
import functools

import jax
import jax.numpy as jnp
from jax import lax
from jax.experimental import pallas as pl
from jax.experimental.pallas import tpu as pltpu
from jax.experimental.pallas import tpu_sc as plsc

N_NODES = 10000
N_GRAPHS = 64
D = 128
E = 320000
BN_EPS = 1e-5

NC = 2
NS = 16
NW = NC * NS
EPW = E // NW
CH = 96
NCHUNK = EPW // CH
TAIL = EPW - NCHUNK * CH
ROWS_PER_TILE = 632
NPAD = ROWS_PER_TILE * NS

_mesh = plsc.VectorSubcoreMesh(
    core_axis_name="c", subcore_axis_name="s", num_cores=NC, num_subcores=NS
)


NBR = 3
NBI = 6
GD = 2
ID = 5
SW = NBR - GD
NSTEADY = (NCHUNK - 1 - ID - 2 + 1) // NBI
STEADY_END = 2 + NBI * NSTEADY


def _make_sc_segsum(with_deg):
  out_type = [jax.ShapeDtypeStruct((NC, NPAD, D), jnp.float32)]
  scratch = (
      [pltpu.VMEM((CH,), jnp.int32) for _ in range(2 * NBI)]
      + [pltpu.VMEM((CH, D), jnp.float32) for _ in range(NBR)]
      + [pltpu.VMEM((TAIL,), jnp.int32) for _ in range(2)]
      + [pltpu.VMEM((TAIL, D), jnp.float32)]
      + [pltpu.VMEM_SHARED((NPAD, D), jnp.float32)]
      + [pltpu.SemaphoreType.DMA for _ in range(NBI + 2 * NBR + 2)]
  )
  if with_deg:
    out_type.append(jax.ShapeDtypeStruct((NC * NPAD,), jnp.float32))
    scratch += [pltpu.VMEM((CH,), jnp.float32),
                pltpu.VMEM((ROWS_PER_TILE,), jnp.float32),
                pltpu.VMEM_SHARED((NPAD,), jnp.float32)]

  @functools.partial(pl.kernel, mesh=_mesh, out_type=out_type,
                     scratch_types=scratch)
  def sc_segsum(h_hbm, src_hbm, dst_hbm, zeros_hbm, *outs_refs):
    if with_deg:
        out_hbm, deg_hbm = outs_refs[:2]
        refs = outs_refs[2:]
        ones_v, dbuf, deg_sh = refs[-3:]
    else:
        (out_hbm,), refs = outs_refs[:1], outs_refs[1:]
    srcs = refs[0:NBI]
    dsts = refs[NBI:2 * NBI]
    rows = refs[2 * NBI:2 * NBI + NBR]
    src_t, dst_t, rows_t = refs[2 * NBI + NBR:2 * NBI + NBR + 3]
    agg_sh = refs[2 * NBI + NBR + 3]
    sems = refs[2 * NBI + NBR + 4:2 * NBI + NBR + 4 + NBI + 2 * NBR + 2]
    isems = sems[0:NBI]
    gsems = sems[NBI:NBI + NBR]
    ssems = sems[NBI + NBR:NBI + 2 * NBR]
    tsem = sems[NBI + 2 * NBR]
    zsem = sems[NBI + 2 * NBR + 1]

    c = lax.axis_index("c")
    s = lax.axis_index("s")
    wid = s * NC + c

    row0 = pl.multiple_of(s * ROWS_PER_TILE, 8)
    zero_desc = pltpu.async_copy(zeros_hbm.at[pl.ds(row0, ROWS_PER_TILE)],
                                 agg_sh.at[pl.ds(row0, ROWS_PER_TILE)], zsem)
    if with_deg:
        for k in range(CH // 16):
            ones_v[pl.ds(16 * k, 16)] = jnp.full((16,), 1.0, jnp.float32)
        for k in range(ROWS_PER_TILE // 16):
            dbuf[pl.ds(16 * k, 16)] = jnp.zeros((16,), jnp.float32)
        pltpu.sync_copy(dbuf, deg_sh.at[pl.ds(row0, ROWS_PER_TILE)])

    base = wid * EPW

    def idx_start(i, b):
        off = pl.multiple_of(base + i * CH, 8)
        pltpu.async_copy(src_hbm.at[pl.ds(off, CH)], srcs[b], isems[b])
        pltpu.async_copy(dst_hbm.at[pl.ds(off, CH)], dsts[b], isems[b])

    def idx_wait(b):
        pltpu.make_async_copy(src_hbm.at[pl.ds(0, CH)], srcs[b], isems[b]).wait()
        pltpu.make_async_copy(dst_hbm.at[pl.ds(0, CH)], dsts[b], isems[b]).wait()

    CH2 = CH // 2

    def gather_start(bi, br):
        pltpu.async_copy(h_hbm.at[srcs[bi].at[pl.ds(0, CH2)]],
                         rows[br].at[pl.ds(0, CH2)], gsems[br])
        pltpu.async_copy(h_hbm.at[srcs[bi].at[pl.ds(CH2, CH2)]],
                         rows[br].at[pl.ds(CH2, CH2)], gsems[br])

    def gather_wait(bi, br):
        pltpu.make_async_copy(h_hbm.at[srcs[bi].at[pl.ds(0, CH2)]],
                              rows[br].at[pl.ds(0, CH2)], gsems[br]).wait()
        pltpu.make_async_copy(h_hbm.at[srcs[bi].at[pl.ds(CH2, CH2)]],
                              rows[br].at[pl.ds(CH2, CH2)], gsems[br]).wait()

    def scatter_start(bi, br):
        pltpu.async_copy(rows[br], agg_sh.at[dsts[bi]], ssems[br], add=True)
        if with_deg:
            pltpu.async_copy(ones_v, deg_sh.at[dsts[bi]], ssems[br], add=True)

    def scatter_wait(bi, br):
        pltpu.make_async_copy(rows[br], agg_sh.at[dsts[bi]], ssems[br]).wait()
        if with_deg:
            pltpu.make_async_copy(ones_v, deg_sh.at[dsts[bi]],
                                  ssems[br]).wait()

    def emit_body(i, r, first=False, last_g=True, last_i=True):
        if last_g:
            idx_wait((r + GD) % NBI)
        if not first:
            scatter_wait((r + GD) % NBI, (r + GD) % NBR)
        if last_g:
            gather_start((r + GD) % NBI, (r + GD) % NBR)
        gather_wait(r, r % NBR)
        scatter_start(r, r % NBR)
        if last_i:
            idx_start(i + ID, (r + ID) % NBI)

    for j in range(ID):
        idx_start(j, j)
    for j in range(GD):
        idx_wait(j)
        gather_start(j, j)
    zero_desc.wait()
    plsc.subcore_barrier()
    for j in range(2):
        emit_body(j, j, first=(j < SW))

    def outer(g, carry):
        i0 = 2 + NBI * g
        for k in range(NBI):
            emit_body(i0 + k, (2 + k) % NBI)
        return carry

    lax.fori_loop(0, NSTEADY, outer, 0)

    for i in range(STEADY_END, NCHUNK):
        emit_body(i, i % NBI, last_g=(i + GD < NCHUNK), last_i=(i + ID < NCHUNK))
    for i in range(NCHUNK - SW, NCHUNK):
        scatter_wait(i % NBI, i % NBR)

    toff = pl.multiple_of(base + NCHUNK * CH, 8)
    pltpu.sync_copy(src_hbm.at[pl.ds(toff, TAIL)], src_t)
    pltpu.sync_copy(dst_hbm.at[pl.ds(toff, TAIL)], dst_t)
    pltpu.async_copy(h_hbm.at[src_t], rows_t, tsem).wait()
    pltpu.sync_copy(rows_t, agg_sh.at[dst_t], add=True)
    if with_deg:
        pltpu.sync_copy(ones_v.at[pl.ds(0, TAIL)], deg_sh.at[dst_t], add=True)

    plsc.subcore_barrier()

    pltpu.sync_copy(agg_sh.at[pl.ds(row0, ROWS_PER_TILE)],
                    out_hbm.at[c, pl.ds(row0, ROWS_PER_TILE)])
    if with_deg:
        doff = pl.multiple_of(c * NPAD + row0, 8)
        pltpu.sync_copy(deg_sh.at[pl.ds(row0, ROWS_PER_TILE)], dbuf)
        pltpu.sync_copy(dbuf, deg_hbm.at[pl.ds(doff, ROWS_PER_TILE)])

  return sc_segsum


_sc_segsum = _make_sc_segsum(False)
_sc_segsum_deg = _make_sc_segsum(True)


R = 1000
GRID = N_NODES // R


def _mlp_stats_body(h_ref, agg_ref, w1_ref, w2_ref, p_ref, stats_ref):
    z = h_ref[...] + agg_ref[0] + agg_ref[1]
    y = jnp.maximum(jnp.dot(z, w1_ref[...], preferred_element_type=jnp.float32), 0.0)
    y = jnp.dot(y, w2_ref[...], preferred_element_type=jnp.float32)
    p = jnp.maximum(y, 0.0)
    p_ref[...] = p
    st = jnp.concatenate(
        [jnp.sum(p, axis=0, keepdims=True),
         jnp.sum(p * p, axis=0, keepdims=True)], axis=0)

    @pl.when(pl.program_id(0) == 0)
    def _():
        stats_ref[...] = jnp.zeros_like(stats_ref)

    stats_ref[...] += st


_mlp_stats = pl.pallas_call(
    _mlp_stats_body,
    grid=(GRID,),
    in_specs=[
        pl.BlockSpec((R, D), lambda i: (i, 0)),
        pl.BlockSpec((NC, R, D), lambda i: (0, i, 0)),
        pl.BlockSpec((D, D), lambda i: (0, 0)),
        pl.BlockSpec((D, D), lambda i: (0, 0)),
    ],
    out_specs=[
        pl.BlockSpec((R, D), lambda i: (i, 0)),
        pl.BlockSpec((2, D), lambda i: (0, 0)),
    ],
    out_shape=[
        jax.ShapeDtypeStruct((N_NODES, D), jnp.float32),
        jax.ShapeDtypeStruct((2, D), jnp.float32),
    ],
)


def _bn_coefs(stats_ref, g_ref, b_ref):
    inv_n = 1.0 / N_NODES
    mean = stats_ref[0] * inv_n
    var = stats_ref[1] * inv_n - mean * mean
    a = g_ref[0] * lax.rsqrt(var + BN_EPS)
    cshift = b_ref[0] - mean * a
    return a, cshift


def _mid_body(p_ref, agg_ref, stats_ref, g_ref, b_ref, d0_ref, d1_ref,
              w1_ref, w2_ref, pout_ref, stats_out_ref, h_ref):
    a, cshift = _bn_coefs(stats_ref, g_ref, b_ref)
    p = p_ref[...]
    h_ref[...] = p * a + cshift
    deg = d0_ref[...] + d1_ref[...]
    z = (p + agg_ref[0] + agg_ref[1]) * a + (1.0 + deg) * cshift
    y = jnp.maximum(jnp.dot(z, w1_ref[...], preferred_element_type=jnp.float32), 0.0)
    y = jnp.dot(y, w2_ref[...], preferred_element_type=jnp.float32)
    pn = jnp.maximum(y, 0.0)
    pout_ref[...] = pn
    st = jnp.concatenate(
        [jnp.sum(pn, axis=0, keepdims=True),
         jnp.sum(pn * pn, axis=0, keepdims=True)], axis=0)

    @pl.when(pl.program_id(0) == 0)
    def _():
        stats_out_ref[...] = jnp.zeros_like(stats_out_ref)

    stats_out_ref[...] += st


_MID_SPECS = [
    pl.BlockSpec((R, D), lambda i: (i, 0)),
    pl.BlockSpec((NC, R, D), lambda i: (0, i, 0)),
    pl.BlockSpec((2, D), lambda i: (0, 0)),
    pl.BlockSpec((1, D), lambda i: (0, 0)),
    pl.BlockSpec((1, D), lambda i: (0, 0)),
    pl.BlockSpec((R, 1), lambda i: (i, 0)),
    pl.BlockSpec((R, 1), lambda i: (i, 0)),
    pl.BlockSpec((D, D), lambda i: (0, 0)),
    pl.BlockSpec((D, D), lambda i: (0, 0)),
]

_mlp_mid = pl.pallas_call(
    _mid_body,
    grid=(GRID,),
    in_specs=_MID_SPECS,
    out_specs=[
        pl.BlockSpec((R, D), lambda i: (i, 0)),
        pl.BlockSpec((2, D), lambda i: (0, 0)),
        pl.BlockSpec((R, D), lambda i: (i, 0)),
    ],
    out_shape=[
        jax.ShapeDtypeStruct((N_NODES, D), jnp.float32),
        jax.ShapeDtypeStruct((2, D), jnp.float32),
        jax.ShapeDtypeStruct((N_NODES, D), jnp.float32),
    ],
)


def _final_body(p_ref, agg_ref, stats_ref, g_ref, b_ref, d0_ref, d1_ref,
                w1_ref, w2_ref, batch_ref, z_ref, pool_ref, h_ref):
    a, cshift = _bn_coefs(stats_ref, g_ref, b_ref)
    p = p_ref[...]
    h_ref[...] = p * a + cshift
    deg = d0_ref[...] + d1_ref[...]
    z = (p + agg_ref[0] + agg_ref[1]) * a + (1.0 + deg) * cshift
    y = jnp.maximum(jnp.dot(z, w1_ref[...], preferred_element_type=jnp.float32), 0.0)
    y = jnp.dot(y, w2_ref[...], preferred_element_type=jnp.float32)
    z_ref[...] = y
    seg = lax.broadcasted_iota(jnp.int32, (N_GRAPHS, 1), 0)
    mask = (batch_ref[0] == seg).astype(jnp.float32)
    part = jnp.dot(mask, y, preferred_element_type=jnp.float32)

    @pl.when(pl.program_id(0) == 0)
    def _():
        pool_ref[...] = jnp.zeros_like(pool_ref)

    pool_ref[...] += part


_final_mlp_pool = pl.pallas_call(
    _final_body,
    grid=(GRID,),
    in_specs=_MID_SPECS + [pl.BlockSpec((1, 1, R), lambda i: (i, 0, 0))],
    out_specs=[
        pl.BlockSpec((R, D), lambda i: (i, 0)),
        pl.BlockSpec((N_GRAPHS, D), lambda i: (0, 0)),
        pl.BlockSpec((R, D), lambda i: (i, 0)),
    ],
    out_shape=[
        jax.ShapeDtypeStruct((N_NODES, D), jnp.float32),
        jax.ShapeDtypeStruct((N_GRAPHS, D), jnp.float32),
        jax.ShapeDtypeStruct((N_NODES, D), jnp.float32),
    ],
)


def kernel(x, edge_index, batch, W1_0, W2_0, g0, b0, W1_1, W2_1, g1, b1,
           W1_2, W2_2):
    src = edge_index[0].astype(jnp.int32)
    dst = edge_index[1].astype(jnp.int32)
    zeros = jnp.zeros((NPAD, D), jnp.float32)
    batch3d = batch.astype(jnp.int32).reshape(GRID, 1, R)
    g0r, b0r = g0.reshape(1, D), b0.reshape(1, D)
    g1r, b1r = g1.reshape(1, D), b1.reshape(1, D)

    agg, deg = _sc_segsum_deg(x, src, dst, zeros)
    d0 = deg[:NPAD].reshape(NPAD, 1)
    d1 = deg[NPAD:].reshape(NPAD, 1)
    p0, stats0 = _mlp_stats(x, agg, W1_0, W2_0)

    agg, = _sc_segsum(p0, src, dst, zeros)
    p1, stats1, h1 = _mlp_mid(p0, agg, stats0, g0r, b0r, d0, d1, W1_1, W2_1)

    agg, = _sc_segsum(p1, src, dst, zeros)
    z3, xpool, h2 = _final_mlp_pool(p1, agg, stats1, g1r, b1r, d0, d1,
                                    W1_2, W2_2, batch3d)

    return (xpool, jnp.concatenate([h1, h2, z3], axis=1))
